# baseline (speedup 1.0000x reference)
import functools

import jax
import jax.numpy as jnp
from jax import lax
from jax.experimental import pallas as pl
from jax.experimental.pallas import tpu as pltpu
from jax.experimental.pallas import tpu_sc as plsc

_N_EMB = 8192
_DIM = 256
_B = 16384
_CHUNK = 128

_info = plsc.get_sparse_core_info()
_NC, _NS = _info.num_cores, _info.num_subcores
_NW = _NC * _NS
_BPW = _B // _NW

_mesh = plsc.VectorSubcoreMesh(core_axis_name="c", subcore_axis_name="s")


@functools.partial(
    pl.kernel,
    mesh=_mesh,
    out_type=jax.ShapeDtypeStruct((_B, _DIM), jnp.float32),
    scratch_types=[
        pltpu.VMEM((_CHUNK,), jnp.int32),
        pltpu.VMEM((_CHUNK, _DIM), jnp.float32),
        pltpu.SemaphoreType.DMA,
    ],
)
def _sc_gather(table_hbm, idx_hbm, out_hbm, idx_v, rows_v, sem):
    wid = lax.axis_index("s") * _NC + lax.axis_index("c")
    base = wid * _BPW

    def chunk(i, carry):
        off = base + i * _CHUNK
        pltpu.sync_copy(idx_hbm.at[pl.ds(off, _CHUNK)], idx_v)
        pltpu.async_copy(table_hbm.at[idx_v], rows_v, sem).wait()
        pltpu.sync_copy(rows_v, out_hbm.at[pl.ds(off, _CHUNK)])
        return carry

    lax.fori_loop(0, _BPW // _CHUNK, chunk, 0)


def kernel(inputs, W):
    flat_inputs = inputs.reshape(-1, _DIM)
    distances = (jnp.sum(flat_inputs ** 2, axis=1, keepdims=True)
                 + jnp.sum(W ** 2, axis=1)
                 - 2.0 * jnp.matmul(flat_inputs, W.T))
    encoding_indices = jnp.argmin(distances, axis=1)
    Wq = W.astype(jnp.bfloat16).astype(jnp.float32)
    icol = lax.iota(jnp.float32, _N_EMB).reshape(_N_EMB, 1)
    Wcat = jnp.concatenate([W, icol], axis=1)
    anchor = jnp.take(Wcat, encoding_indices, axis=0)
    idx_sc = anchor[:, _DIM].astype(jnp.int32)
    quantized = _sc_gather(Wq, idx_sc).reshape(inputs.shape)
    return (quantized, encoding_indices)


# speedup vs baseline: 1.0092x; 1.0092x over previous
import functools

import jax
import jax.numpy as jnp
from jax import lax
from jax.experimental import pallas as pl
from jax.experimental.pallas import tpu as pltpu
from jax.experimental.pallas import tpu_sc as plsc

_N_EMB = 8192
_DIM = 256
_B = 16384
_CHUNK = 128

_info = plsc.get_sparse_core_info()
_NC, _NS = _info.num_cores, _info.num_subcores
_NW = _NC * _NS
_BPW = _B // _NW

_mesh = plsc.VectorSubcoreMesh(core_axis_name="c", subcore_axis_name="s")


@functools.partial(
    pl.kernel,
    mesh=_mesh,
    out_type=jax.ShapeDtypeStruct((_B, _DIM), jnp.float32),
    scratch_types=[
        pltpu.VMEM((_CHUNK,), jnp.int32),
        pltpu.VMEM((_CHUNK, _DIM), jnp.float32),
        pltpu.SemaphoreType.DMA,
    ],
)
def _sc_gather(table_hbm, idx_hbm, out_hbm, idx_v, rows_v, sem):
    wid = lax.axis_index("s") * _NC + lax.axis_index("c")
    base = wid * _BPW

    def chunk(i, carry):
        off = base + i * _CHUNK
        pltpu.sync_copy(idx_hbm.at[pl.ds(off, _CHUNK)], idx_v)
        pltpu.async_copy(table_hbm.at[idx_v], rows_v, sem).wait()
        pltpu.sync_copy(rows_v, out_hbm.at[pl.ds(off, _CHUNK)])
        return carry

    lax.fori_loop(0, _BPW // _CHUNK, chunk, 0)


def kernel(inputs, W):
    flat_inputs = inputs.reshape(-1, _DIM)
    distances = (jnp.sum(flat_inputs ** 2, axis=1, keepdims=True)
                 + jnp.sum(W ** 2, axis=1)
                 - 2.0 * jnp.matmul(flat_inputs, W.T))
    encoding_indices = jnp.argmin(distances, axis=1)
    icol = lax.iota(jnp.float32, _N_EMB).reshape(_N_EMB, 1)
    Wcat = jnp.concatenate([W, icol], axis=1)
    anchor = jnp.take(Wcat, encoding_indices, axis=0)
    idx_sc = anchor[:, _DIM].astype(jnp.int32)
    quantized = _sc_gather(W, idx_sc).reshape(inputs.shape)
    return (quantized, encoding_indices)
